# baseline (speedup 1.0000x reference)
import jax
import jax.numpy as jnp
from jax import lax
from jax.experimental import pallas as pl
from jax.experimental.pallas import tpu as pltpu
from jax.experimental.pallas import tpu_sc as plsc


def _sc_expand(c_sh):
    nh = 16
    n = 32

    def body(c_hbm, out_hbm, s_v, sem):
        cid = lax.axis_index("c")
        sid = lax.axis_index("s")
        wid = sid * 2 + cid
        h = wid // 2
        half = wid % 2
        builds = [
            pltpu.async_copy(
                c_hbm.at[(31 - pw) % 8, h, :, pl.ds((31 - pw) - (31 - pw) % 8, n)],
                s_v.at[pw],
                sem,
            )
            for pw in range(n)
        ]
        for b in builds:
            b.wait()
        bands = []
        for i in range(n // 2):
            ph = half * (n // 2) + i
            bands.append(
                pltpu.async_copy(
                    s_v.at[:, pl.ds(31 - ph, n), :],
                    out_hbm.at[h, pl.ds(ph * n, n)],
                    sem,
                )
            )
        for b in bands:
            b.wait()

    run = pl.kernel(
        body,
        out_type=jax.ShapeDtypeStruct((nh, n * n, n, n), jnp.float32),
        mesh=plsc.VectorSubcoreMesh(core_axis_name="c", subcore_axis_name="s"),
        scratch_types=[
            pltpu.VMEM((n, 63, n), jnp.float32),
            pltpu.SemaphoreType.DMA,
        ],
        compiler_params=pltpu.CompilerParams(use_tc_tiling_on_sc=False),
    )
    return run(c_sh)


def kernel(H, W, relative_position_bias_table):
    table = relative_position_bias_table
    nh = table.shape[1]
    side = int(round(table.shape[0] ** 0.5))
    n = (side + 1) // 2
    off = (jnp.asarray(H, jnp.int32) - n) + (jnp.asarray(W, jnp.int32) - n)
    table = jnp.roll(table, -off, axis=0)
    c = jnp.transpose(table.reshape(side, side, nh)[::-1, ::-1, :], (2, 0, 1))
    c_wide = jnp.pad(c, ((0, 0), (0, 0), (0, 72 - side)))
    c_sh = jnp.stack([c_wide[:, :, s:s + 64] for s in range(8)])
    out = _sc_expand(c_sh)
    return out.reshape(1, nh, n * n, n * n)


# speedup vs baseline: 1.8586x; 1.8586x over previous
import jax
import jax.numpy as jnp
from jax import lax
from jax.experimental import pallas as pl
from jax.experimental.pallas import tpu as pltpu
from jax.experimental.pallas import tpu_sc as plsc


def _sc_expand(c_shift):
    nh = 16
    n = 32

    def body(c_hbm, out_hbm, s_v, sem):
        cid = lax.axis_index("c")
        sid = lax.axis_index("s")
        wid = sid * 2 + cid
        h = wid // 2
        half = wid % 2
        builds = [
            pltpu.async_copy(c_hbm.at[31 - pw, h], s_v.at[pw], sem)
            for pw in range(n)
        ]
        for b in builds:
            b.wait()
        s2 = s_v
        bands = []
        for i in range(n // 2):
            ph = half * (n // 2) + i
            bands.append(
                pltpu.async_copy(
                    s2.at[:, pl.ds((31 - ph) * n, n * n)],
                    out_hbm.at[0, h, pl.ds(ph * n, n), :],
                    sem,
                )
            )
        for b in bands:
            b.wait()

    run = pl.kernel(
        body,
        out_type=jax.ShapeDtypeStruct((1, nh, n * n, n * n), jnp.float32),
        mesh=plsc.VectorSubcoreMesh(core_axis_name="c", subcore_axis_name="s"),
        scratch_types=[
            pltpu.VMEM((n, 63 * n), jnp.float32),
            pltpu.SemaphoreType.DMA,
        ],
        compiler_params=pltpu.CompilerParams(use_tc_tiling_on_sc=False),
    )
    return run(c_shift)


def kernel(H, W, relative_position_bias_table):
    table = relative_position_bias_table
    nh = table.shape[1]
    side = int(round(table.shape[0] ** 0.5))
    n = (side + 1) // 2
    off = (jnp.asarray(H, jnp.int32) - n) + (jnp.asarray(W, jnp.int32) - n)
    table = jnp.roll(table, -off, axis=0)
    c = jnp.transpose(table.reshape(side, side, nh)[::-1, ::-1, :], (2, 0, 1))
    c_shift = jnp.stack([c[:, :, t:t + n] for t in range(n)])
    return _sc_expand(c_shift.reshape(n, nh, (2 * n - 1) * n))


# speedup vs baseline: 2.2965x; 1.2356x over previous
import numpy as np
import jax
import jax.numpy as jnp
from jax import lax
from jax.experimental import pallas as pl
from jax.experimental.pallas import tpu as pltpu
from jax.experimental.pallas import tpu_sc as plsc


def _window_selector(side, n):
    b = np.arange(side)[:, None, None]
    t = np.arange(n)[None, :, None]
    w = np.arange(n)[None, None, :]
    return jnp.asarray((b == t + w).astype(np.float32))


def _sc_expand(c_flat, nh, n):
    row = 60 * n
    hw = n * n

    def body(c_hbm, out_hbm, s4, sem):
        cid = lax.axis_index("c")
        sid = lax.axis_index("s")
        wid = sid * 2 + cid
        h = wid // 2
        half = wid % 2
        builds = []
        for i in range(n // 2):
            pw = half * (n // 2) + i
            for k in range(4):
                src_off = pl.multiple_of(((h * 4 + k) * n + (31 - pw)) * row, row)
                builds.append(
                    pltpu.async_copy(
                        c_hbm.at[pl.ds(src_off, row)],
                        s4.at[i, pl.ds(k * row, row)],
                        sem,
                    )
                )
        for b in builds:
            b.wait()
        bands = []
        for ph in range(n):
            tb = 31 - ph
            col = (tb % 4) * row + (tb // 4) * 128
            p0 = pl.multiple_of(ph * n + half * (n // 2), n // 2)
            bands.append(
                pltpu.async_copy(
                    s4.at[:, pl.ds(col, hw)],
                    out_hbm.at[0, h, pl.ds(p0, n // 2), :],
                    sem,
                )
            )
        for b in bands:
            b.wait()

    run = pl.kernel(
        body,
        out_type=jax.ShapeDtypeStruct((1, nh, hw, hw), jnp.float32),
        mesh=plsc.VectorSubcoreMesh(core_axis_name="c", subcore_axis_name="s"),
        scratch_types=[
            pltpu.VMEM((n // 2, 4 * row), jnp.float32),
            pltpu.SemaphoreType.DMA,
        ],
    )
    return run(c_flat)


def kernel(H, W, relative_position_bias_table):
    table = relative_position_bias_table
    nh = table.shape[1]
    side = int(round(table.shape[0] ** 0.5))
    n = (side + 1) // 2
    off = (jnp.asarray(H, jnp.int32) - n) + (jnp.asarray(W, jnp.int32) - n)
    table = jnp.roll(table, -off, axis=0)
    c = jnp.transpose(table.reshape(side, side, nh)[::-1, ::-1, :], (2, 0, 1))
    strip_e = jnp.einsum('hrb,btw->htrw', c, _window_selector(side, n),
                         preferred_element_type=jnp.float32,
                         precision=jax.lax.Precision.HIGHEST)
    c4e = jnp.stack([strip_e[:, :, k:k + 60, :] for k in range(4)], axis=1)
    return _sc_expand(c4e.reshape(-1), nh, n)
